# Initial kernel scaffold; baseline (speedup 1.0000x reference)
import functools

import jax
import jax.numpy as jnp
from jax import lax
from jax.experimental import pallas as pl
from jax.experimental.pallas import tpu as pltpu
from jax.experimental.pallas import tpu_sc as plsc

NUM_CODES = 8192
DIM = 64
N_TOKENS = 16 * 576
TOK_TILE = 256
COMMITMENT_COST = 0.25

SC_CORES = 2
SC_SUBCORES = 16
SC_WORKERS = SC_CORES * SC_SUBCORES
TOK_PER_WORKER = N_TOKENS // SC_WORKERS


def _dist_argmin_body(x_ref, cbt_ref, idx_ref, loss_ref):
    x = x_ref[...]
    cbt = cbt_ref[...]
    xsq = jnp.sum(x * x, axis=1, keepdims=True)
    esq = jnp.sum(cbt * cbt, axis=0, keepdims=True)
    mm = jnp.dot(x, cbt)
    dist = (xsq + esq) - 2.0 * mm
    mind = jnp.min(dist, axis=1, keepdims=True)
    ii = lax.broadcasted_iota(jnp.int32, dist.shape, 1)
    idx = jnp.min(jnp.where(dist == mind, ii, NUM_CODES),
                  axis=1, keepdims=True)
    idx_ref[...] = idx

    @pl.when(pl.program_id(0) == 0)
    def _init():
        loss_ref[0, 0] = 0.0

    loss_ref[0, 0] += jnp.sum(mind)

    @pl.when(pl.program_id(0) == pl.num_programs(0) - 1)
    def _finish():
        loss_ref[0, 0] = loss_ref[0, 0] * (COMMITMENT_COST / (N_TOKENS * DIM))


def _distances_argmin(xf, cbt):
    return pl.pallas_call(
        _dist_argmin_body,
        grid=(N_TOKENS // TOK_TILE,),
        in_specs=[
            pl.BlockSpec((TOK_TILE, DIM), lambda i: (i, 0)),
            pl.BlockSpec((DIM, NUM_CODES), lambda i: (0, 0)),
        ],
        out_specs=[
            pl.BlockSpec((TOK_TILE, 1), lambda i: (i, 0)),
            pl.BlockSpec(memory_space=pltpu.SMEM),
        ],
        out_shape=[
            jax.ShapeDtypeStruct((N_TOKENS, 1), jnp.int32),
            jax.ShapeDtypeStruct((1, 1), jnp.float32),
        ],
    )(xf, cbt)


_SC_MESH = plsc.VectorSubcoreMesh(core_axis_name="c", subcore_axis_name="s")


@functools.partial(
    pl.kernel,
    mesh=_SC_MESH,
    out_type=jax.ShapeDtypeStruct((N_TOKENS, DIM), jnp.float32),
    scratch_types=[
        pltpu.VMEM((TOK_PER_WORKER,), jnp.int32),
        pltpu.VMEM((TOK_PER_WORKER, DIM), jnp.float32),
        pltpu.SemaphoreType.DMA,
    ],
)
def _sc_gather(cb_hbm, idx_hbm, out_hbm, idx_v, rows_v, sem):
    wid = lax.axis_index("s") * SC_CORES + lax.axis_index("c")
    base = wid * TOK_PER_WORKER
    pltpu.sync_copy(idx_hbm.at[pl.ds(base, TOK_PER_WORKER)], idx_v)
    pltpu.async_copy(cb_hbm.at[idx_v], rows_v, sem).wait()
    pltpu.sync_copy(rows_v, out_hbm.at[pl.ds(base, TOK_PER_WORKER)])


def kernel(x, codebook):
    input_shape = x.shape
    xf = x.reshape(-1, DIM)
    cbt = codebook.T
    idx2, loss = _distances_argmin(xf, cbt)
    idx_flat = idx2.reshape(-1)
    quantized = _sc_gather(codebook, idx_flat)
    return (quantized.reshape(input_shape),
            idx_flat.reshape(input_shape[:-1]),
            loss.reshape(()))


# speedup vs baseline: 1.0535x; 1.0535x over previous
import functools

import jax
import jax.numpy as jnp
from jax import lax
from jax.experimental import pallas as pl
from jax.experimental.pallas import tpu as pltpu
from jax.experimental.pallas import tpu_sc as plsc

NUM_CODES = 8192
DIM = 64
N_TOKENS = 16 * 576
TOK_TILE = 256
COMMITMENT_COST = 0.25

SC_CORES = 2
SC_SUBCORES = 16
SC_WORKERS = SC_CORES * SC_SUBCORES
TOK_PER_WORKER = N_TOKENS // SC_WORKERS


WINDOW = 2048
N_WINDOWS = NUM_CODES // WINDOW


def _half_argmin(dist):
    m = jnp.min(dist, axis=1, keepdims=True)
    ii = lax.broadcasted_iota(jnp.int32, dist.shape, 1)
    i = jnp.min(jnp.where(dist == m, ii, NUM_CODES), axis=1, keepdims=True)
    return m, i


def _bf16_round(v):
    return v.astype(jnp.bfloat16).astype(jnp.float32)


def _dist_argmin_body(x_ref, xsq_ref, cbt_ref, idx_ref, loss_ref):
    x = x_ref[...]
    cbt = cbt_ref[...]
    xsq = xsq_ref[...]
    esq = jnp.sum(cbt * cbt, axis=0, keepdims=True)
    xb = x.astype(jnp.bfloat16)
    dn = (((1,), (0,)), ((), ()))
    mm = lax.dot_general(xb, cbt, dn, preferred_element_type=jnp.float32)
    dist = (xsq + esq) - 2.0 * mm
    m0, i0 = _half_argmin(dist[:, :WINDOW])
    sel_idx, sel_val = i0, m0
    carry = _bf16_round(m0)
    for w in range(1, N_WINDOWS):
        mw, iw = _half_argmin(dist[:, w * WINDOW:(w + 1) * WINDOW])
        upd = mw < carry
        sel_idx = jnp.where(upd, iw + w * WINDOW, sel_idx)
        sel_val = jnp.where(upd, mw, sel_val)
        carry = _bf16_round(jnp.where(upd, mw, carry))
    idx_ref[...] = sel_idx

    @pl.when(pl.program_id(0) == 0)
    def _init():
        loss_ref[0, 0] = 0.0

    loss_ref[0, 0] += jnp.sum(sel_val)

    @pl.when(pl.program_id(0) == pl.num_programs(0) - 1)
    def _finish():
        loss_ref[0, 0] = loss_ref[0, 0] * (COMMITMENT_COST / (N_TOKENS * DIM))


def _distances_argmin(xf, xsq, cbt):
    return pl.pallas_call(
        _dist_argmin_body,
        grid=(N_TOKENS // TOK_TILE,),
        in_specs=[
            pl.BlockSpec((TOK_TILE, DIM), lambda i: (i, 0)),
            pl.BlockSpec((TOK_TILE, 1), lambda i: (i, 0)),
            pl.BlockSpec((DIM, NUM_CODES), lambda i: (0, 0)),
        ],
        out_specs=[
            pl.BlockSpec((TOK_TILE, 1), lambda i: (i, 0)),
            pl.BlockSpec(memory_space=pltpu.SMEM),
        ],
        out_shape=[
            jax.ShapeDtypeStruct((N_TOKENS, 1), jnp.int32),
            jax.ShapeDtypeStruct((1, 1), jnp.float32),
        ],
    )(xf, xsq, cbt)


GATHER_CHUNK = 96
N_CHUNKS = TOK_PER_WORKER // GATHER_CHUNK
PAD_DIM = 128


def _make_sc_gather():
    mesh = plsc.VectorSubcoreMesh(core_axis_name="c", subcore_axis_name="s")

    @functools.partial(
        pl.kernel,
        mesh=mesh,
        out_type=jax.ShapeDtypeStruct((N_TOKENS, PAD_DIM), jnp.float32),
        scratch_types=[
            [pltpu.VMEM((GATHER_CHUNK,), jnp.int32) for _ in range(N_CHUNKS)],
            [pltpu.VMEM((GATHER_CHUNK, PAD_DIM), jnp.float32)
             for _ in range(N_CHUNKS)],
            pltpu.SemaphoreType.DMA,
        ],
    )
    def _sc_gather(cb_hbm, idx_hbm, out_hbm, idx_vs, rows_vs, sem):
        wid = lax.axis_index("s") * SC_CORES + lax.axis_index("c")
        base = wid * TOK_PER_WORKER
        for j in range(N_CHUNKS):
            pltpu.sync_copy(idx_hbm.at[pl.ds(base + j * GATHER_CHUNK,
                                             GATHER_CHUNK)], idx_vs[j])
        copies = [pltpu.async_copy(cb_hbm.at[idx_vs[j]], rows_vs[j], sem)
                  for j in range(N_CHUNKS)]
        for j in range(N_CHUNKS):
            copies[j].wait()
            pltpu.sync_copy(rows_vs[j],
                            out_hbm.at[pl.ds(base + j * GATHER_CHUNK,
                                             GATHER_CHUNK)])

    return _sc_gather


def kernel(x, codebook):
    input_shape = x.shape
    xf = x.reshape(-1, DIM)
    cbt = codebook.T
    xsq = jnp.sum(xf ** 2, axis=1, keepdims=True)
    idx2, loss = _distances_argmin(xf, xsq, cbt)
    idx_flat = idx2.reshape(-1)
    cb_pad = jnp.concatenate([codebook, jnp.zeros_like(codebook)], axis=1)
    quantized = _make_sc_gather()(cb_pad, idx_flat)[:, :DIM]
    return (quantized.reshape(input_shape),
            idx_flat.reshape(input_shape[:-1]),
            loss.reshape(()))


# speedup vs baseline: 1.1284x; 1.0711x over previous
import functools

import jax
import jax.numpy as jnp
from jax import lax
from jax.experimental import pallas as pl
from jax.experimental.pallas import tpu as pltpu
from jax.experimental.pallas import tpu_sc as plsc

NUM_CODES = 8192
DIM = 64
N_TOKENS = 16 * 576
TOK_TILE = 512
COMMITMENT_COST = 0.25

SC_CORES = 2
SC_SUBCORES = 16
SC_WORKERS = SC_CORES * SC_SUBCORES
TOK_PER_WORKER = N_TOKENS // SC_WORKERS


WINDOW = 2048
N_WINDOWS = NUM_CODES // WINDOW


def _half_argmin(dist):
    m = jnp.min(dist, axis=1, keepdims=True)
    ii = lax.broadcasted_iota(jnp.int32, dist.shape, 1)
    i = jnp.min(jnp.where(dist == m, ii, NUM_CODES), axis=1, keepdims=True)
    return m, i


def _bf16_round(v):
    return v.astype(jnp.bfloat16).astype(jnp.float32)


def _dist_argmin_body(x_ref, xsq_ref, esq_ref, cbt_ref, idx_ref, loss_ref):
    x = x_ref[...]
    cbt = cbt_ref[...]
    xsq = xsq_ref[...]
    esq = esq_ref[...]
    xb = x.astype(jnp.bfloat16)
    dn = (((1,), (0,)), ((), ()))
    mm = lax.dot_general(xb, cbt, dn, preferred_element_type=jnp.float32)
    dist = (xsq + esq) - 2.0 * mm
    m0, i0 = _half_argmin(dist[:, :WINDOW])
    sel_idx, sel_val = i0, m0
    carry = _bf16_round(m0)
    for w in range(1, N_WINDOWS):
        mw, iw = _half_argmin(dist[:, w * WINDOW:(w + 1) * WINDOW])
        upd = mw < carry
        sel_idx = jnp.where(upd, iw + w * WINDOW, sel_idx)
        sel_val = jnp.where(upd, mw, sel_val)
        carry = _bf16_round(jnp.where(upd, mw, carry))
    idx_ref[...] = sel_idx

    @pl.when(pl.program_id(0) == 0)
    def _init():
        loss_ref[0, 0] = 0.0

    loss_ref[0, 0] += jnp.sum(sel_val)

    @pl.when(pl.program_id(0) == pl.num_programs(0) - 1)
    def _finish():
        loss_ref[0, 0] = loss_ref[0, 0] * (COMMITMENT_COST / (N_TOKENS * DIM))


def _distances_argmin(xf, xsq, esq, cbt):
    return pl.pallas_call(
        _dist_argmin_body,
        grid=(N_TOKENS // TOK_TILE,),
        in_specs=[
            pl.BlockSpec((TOK_TILE, DIM), lambda i: (i, 0)),
            pl.BlockSpec((TOK_TILE, 1), lambda i: (i, 0)),
            pl.BlockSpec((1, NUM_CODES), lambda i: (0, 0)),
            pl.BlockSpec((DIM, NUM_CODES), lambda i: (0, 0)),
        ],
        out_specs=[
            pl.BlockSpec((TOK_TILE, 1), lambda i: (i, 0)),
            pl.BlockSpec(memory_space=pltpu.SMEM),
        ],
        out_shape=[
            jax.ShapeDtypeStruct((N_TOKENS, 1), jnp.int32),
            jax.ShapeDtypeStruct((1, 1), jnp.float32),
        ],
    )(xf, xsq, esq, cbt)


GATHER_CHUNK = 96
N_CHUNKS = TOK_PER_WORKER // GATHER_CHUNK
PAD_DIM = 128


def _make_sc_gather():
    mesh = plsc.VectorSubcoreMesh(core_axis_name="c", subcore_axis_name="s")

    @functools.partial(
        pl.kernel,
        mesh=mesh,
        out_type=jax.ShapeDtypeStruct((N_TOKENS, PAD_DIM), jnp.float32),
        scratch_types=[
            [pltpu.VMEM((GATHER_CHUNK,), jnp.int32) for _ in range(N_CHUNKS)],
            [pltpu.VMEM((GATHER_CHUNK, PAD_DIM), jnp.float32)
             for _ in range(N_CHUNKS)],
            pltpu.SemaphoreType.DMA,
        ],
    )
    def _sc_gather(cb_hbm, idx_hbm, out_hbm, idx_vs, rows_vs, sem):
        wid = lax.axis_index("s") * SC_CORES + lax.axis_index("c")
        base = wid * TOK_PER_WORKER
        for j in range(N_CHUNKS):
            pltpu.sync_copy(idx_hbm.at[pl.ds(base + j * GATHER_CHUNK,
                                             GATHER_CHUNK)], idx_vs[j])
        copies = [pltpu.async_copy(cb_hbm.at[idx_vs[j]], rows_vs[j], sem)
                  for j in range(N_CHUNKS)]
        for j in range(N_CHUNKS):
            copies[j].wait()
            pltpu.sync_copy(rows_vs[j],
                            out_hbm.at[pl.ds(base + j * GATHER_CHUNK,
                                             GATHER_CHUNK)])

    return _sc_gather


def kernel(x, codebook):
    input_shape = x.shape
    xf = x.reshape(-1, DIM)
    cbt = codebook.T
    xsq = jnp.sum(xf ** 2, axis=1, keepdims=True)
    esq = jnp.sum(codebook ** 2, axis=1)[None, :]
    idx2, loss = _distances_argmin(xf, xsq, esq, cbt)
    idx_flat = idx2.reshape(-1)
    cb_pad = jnp.concatenate([codebook, jnp.zeros_like(codebook)], axis=1)
    quantized = _make_sc_gather()(cb_pad, idx_flat)[:, :DIM]
    return (quantized.reshape(input_shape),
            idx_flat.reshape(input_shape[:-1]),
            loss.reshape(()))


# speedup vs baseline: 1.1673x; 1.0345x over previous
import functools

import jax
import jax.numpy as jnp
from jax import lax
from jax.experimental import pallas as pl
from jax.experimental.pallas import tpu as pltpu
from jax.experimental.pallas import tpu_sc as plsc

NUM_CODES = 8192
DIM = 64
N_TOKENS = 16 * 576
TOK_TILE = 512
COMMITMENT_COST = 0.25

SC_CORES = 2
SC_SUBCORES = 16
SC_WORKERS = SC_CORES * SC_SUBCORES
TOK_PER_WORKER = N_TOKENS // SC_WORKERS


WINDOW = 2048
N_WINDOWS = NUM_CODES // WINDOW


def _half_argmin(dist):
    m = jnp.min(dist, axis=1, keepdims=True)
    ii = lax.broadcasted_iota(jnp.int32, dist.shape, 1)
    i = jnp.min(jnp.where(dist == m, ii, NUM_CODES), axis=1, keepdims=True)
    return m, i


def _bf16_round(v):
    return v.astype(jnp.bfloat16).astype(jnp.float32)


def _dist_argmin_body(x_ref, xsq_ref, esq_ref, cbt_ref, idx_ref, loss_ref):
    x = x_ref[...]
    cbt = cbt_ref[...]
    xsq = xsq_ref[...]
    esq = esq_ref[...]
    xb2 = (2.0 * x).astype(jnp.bfloat16)
    dn = (((1,), (0,)), ((), ()))
    mm2 = lax.dot_general(xb2, cbt, dn, preferred_element_type=jnp.float32)
    dist = (xsq + esq) - mm2
    m0, i0 = _half_argmin(dist[:, :WINDOW])
    sel_idx, sel_val = i0, m0
    carry = _bf16_round(m0)
    for w in range(1, N_WINDOWS):
        mw, iw = _half_argmin(dist[:, w * WINDOW:(w + 1) * WINDOW])
        upd = mw < carry
        sel_idx = jnp.where(upd, iw + w * WINDOW, sel_idx)
        sel_val = jnp.where(upd, mw, sel_val)
        carry = _bf16_round(jnp.where(upd, mw, carry))
    idx_ref[...] = sel_idx

    @pl.when(pl.program_id(0) == 0)
    def _init():
        loss_ref[0, 0] = 0.0

    loss_ref[0, 0] += jnp.sum(sel_val)

    @pl.when(pl.program_id(0) == pl.num_programs(0) - 1)
    def _finish():
        loss_ref[0, 0] = loss_ref[0, 0] * (COMMITMENT_COST / (N_TOKENS * DIM))


def _distances_argmin(xf, xsq, esq, cbt):
    return pl.pallas_call(
        _dist_argmin_body,
        grid=(N_TOKENS // TOK_TILE,),
        in_specs=[
            pl.BlockSpec((TOK_TILE, DIM), lambda i: (i, 0)),
            pl.BlockSpec((TOK_TILE, 1), lambda i: (i, 0)),
            pl.BlockSpec((1, NUM_CODES), lambda i: (0, 0)),
            pl.BlockSpec((DIM, NUM_CODES), lambda i: (0, 0)),
        ],
        out_specs=[
            pl.BlockSpec((TOK_TILE, 1), lambda i: (i, 0)),
            pl.BlockSpec(memory_space=pltpu.SMEM),
        ],
        out_shape=[
            jax.ShapeDtypeStruct((N_TOKENS, 1), jnp.int32),
            jax.ShapeDtypeStruct((1, 1), jnp.float32),
        ],
    )(xf, xsq, esq, cbt)


GATHER_CHUNK = 96
N_CHUNKS = TOK_PER_WORKER // GATHER_CHUNK
PAD_DIM = 128


def _make_sc_gather():
    mesh = plsc.VectorSubcoreMesh(core_axis_name="c", subcore_axis_name="s")

    @functools.partial(
        pl.kernel,
        mesh=mesh,
        out_type=jax.ShapeDtypeStruct((N_TOKENS, PAD_DIM), jnp.float32),
        scratch_types=[
            [pltpu.VMEM((GATHER_CHUNK,), jnp.int32) for _ in range(N_CHUNKS)],
            [pltpu.VMEM((GATHER_CHUNK, PAD_DIM), jnp.float32)
             for _ in range(N_CHUNKS)],
            pltpu.SemaphoreType.DMA,
        ],
    )
    def _sc_gather(cb_hbm, idx_hbm, out_hbm, idx_vs, rows_vs, sem):
        wid = lax.axis_index("s") * SC_CORES + lax.axis_index("c")
        base = wid * TOK_PER_WORKER
        for j in range(N_CHUNKS):
            pltpu.sync_copy(idx_hbm.at[pl.ds(base + j * GATHER_CHUNK,
                                             GATHER_CHUNK)], idx_vs[j])
        copies = [pltpu.async_copy(cb_hbm.at[idx_vs[j]], rows_vs[j], sem)
                  for j in range(N_CHUNKS)]
        for j in range(N_CHUNKS):
            copies[j].wait()
            pltpu.sync_copy(rows_vs[j],
                            out_hbm.at[pl.ds(base + j * GATHER_CHUNK,
                                             GATHER_CHUNK)])

    return _sc_gather


def kernel(x, codebook):
    input_shape = x.shape
    xf = x.reshape(-1, DIM)
    cbt = codebook.T
    xsq = jnp.sum(xf ** 2, axis=1, keepdims=True)
    esq = jnp.sum(codebook ** 2, axis=1)[None, :]
    idx2, loss = _distances_argmin(xf, xsq, esq, cbt)
    idx_flat = idx2.reshape(-1)
    cb_pad = jnp.concatenate([codebook, jnp.zeros_like(codebook)], axis=1)
    quantized = _make_sc_gather()(cb_pad, idx_flat)[:, :DIM]
    return (quantized.reshape(input_shape),
            idx_flat.reshape(input_shape[:-1]),
            loss.reshape(()))
